# baseline (speedup 1.0000x reference)
import functools

import jax
import jax.numpy as jnp
from jax import lax
from jax.experimental import pallas as pl
from jax.experimental.pallas import tpu as pltpu
from jax.experimental.pallas import tpu_sc as plsc

B = 16384
F = 26
V = 100000
D = 16
N_DENSE = 13
SP = F * D

NW = 32
ROWS_PER_W = SP // NW
OH = B // 2
UNROLL = 16


def _gather_body(row_base, rows_per_w, idxT_hbm, table_hbm, out_hbm, idx_v, row_v, out_v):
    wid = lax.axis_index("s") * 2 + lax.axis_index("c")
    r0 = row_base + wid * rows_per_w

    def do_row(j, f_prev):
        r = r0 + j
        f = r // D

        @pl.when(f != f_prev)
        def _stage_idx():
            pltpu.sync_copy(idxT_hbm.at[f], idx_v)

        pltpu.sync_copy(table_hbm.at[r], row_v)

        def do_half(h, carry2):
            @plsc.parallel_loop(0, OH, step=16, unroll=UNROLL)
            def gblk(i):
                iv = idx_v[pl.ds(h * OH + i, 16)]
                out_v[pl.ds(i, 16)] = plsc.load_gather(row_v, [iv])

            pltpu.sync_copy(out_v, out_hbm.at[r - row_base, pl.ds(h * OH, OH)])
            return carry2

        lax.fori_loop(0, 2, do_half, 0)
        return f

    lax.fori_loop(0, rows_per_w, do_row, jnp.int32(-1))


@functools.cache
def _gather(row_base, nrows):
    return pl.kernel(
        functools.partial(_gather_body, row_base, nrows // NW),
        out_type=jax.ShapeDtypeStruct((nrows, B), jnp.float32),
        mesh=plsc.VectorSubcoreMesh(core_axis_name="c", subcore_axis_name="s"),
        scratch_types=[
            pltpu.VMEM((B,), jnp.int32),
            pltpu.VMEM((V,), jnp.float32),
            pltpu.VMEM((OH,), jnp.float32),
        ],
        compiler_params=pltpu.CompilerParams(needs_layout_passes=False),
    )


BB = 4096
SPA = 224
SPB = SP - SPA

_PREC = lax.Precision.DEFAULT


def _dott(a, b):
    return lax.dot_general(a, b, (((0,), (0,)), ((), ())),
                           preferred_element_type=jnp.float32, precision=_PREC)


def _dotn(a, b):
    return lax.dot_general(a, b, (((1,), (0,)), ((), ())),
                           preferred_element_type=jnp.float32, precision=_PREC)


def _sel(n):
    ci = lax.broadcasted_iota(jnp.int32, (D, n), 0)
    cj = lax.broadcasted_iota(jnp.int32, (D, n), 1)
    return jnp.where((cj % D) == ci, 1.0, 0.0).astype(jnp.float32)


def _tc1_body(xs_ref, w1_ref, wfmt_ref, p1_ref, s1_ref, q_ref, f1_ref):
    xs = xs_ref[...]
    p1_ref[...] = _dott(w1_ref[...][:SPA], xs).astype(jnp.bfloat16)
    s1_ref[...] = _dotn(_sel(SPA), xs)
    q_ref[...] = jnp.sum(xs * xs, axis=0, keepdims=True)
    f1_ref[...] = _dotn(wfmt_ref[...][:, :SPA], xs)


def _tc2_body(xs_ref, xd_ref, p1_ref, s1a_ref, qa_ref, f1a_ref,
              w1_ref, b1_ref, w2_ref, b2_ref, w3t_ref, b3_ref,
              wdt_ref, bd_ref, wfmt_ref, bfm_ref, o_ref):
    xs = xs_ref[...]
    xd = xd_ref[...]
    s1 = s1a_ref[...] + _dotn(_sel(SPB), xs)
    fm2 = 0.5 * (jnp.sum(s1 * s1, axis=0, keepdims=True) - qa_ref[...]
                 - jnp.sum(xs * xs, axis=0, keepdims=True))
    wfmt = wfmt_ref[...]
    fm1 = (f1a_ref[...] + _dotn(wfmt[:, SPA:SP], xs)
           + _dotn(wfmt[:, SP:], xd) + bfm_ref[...])
    w1 = w1_ref[...]
    h = jnp.maximum(p1_ref[...].astype(jnp.float32) + _dott(w1[SPA:SP], xs)
                    + _dott(w1[SP:], xd) + b1_ref[...], 0.0)
    h = jnp.maximum(_dott(w2_ref[...], h) + b2_ref[...], 0.0)
    h = jnp.maximum(_dotn(w3t_ref[...], h) + b3_ref[...], 0.0)
    dnn = _dotn(wdt_ref[...], h) + bd_ref[...]
    o_ref[...] = jax.nn.sigmoid(fm1 + fm2 + dnn)


def _full(shape):
    return pl.BlockSpec(shape, lambda i: tuple(0 for _ in shape))


def _col(rows):
    return pl.BlockSpec((rows, BB), lambda i: (0, i))


_tc1_call = pl.pallas_call(
    _tc1_body,
    grid=(B // BB,),
    in_specs=[
        _col(SPA),
        _full((SP + N_DENSE, 256)),
        _full((1, SP + N_DENSE)),
    ],
    out_specs=[_col(256), _col(D), _col(1), _col(1)],
    out_shape=[
        jax.ShapeDtypeStruct((256, B), jnp.bfloat16),
        jax.ShapeDtypeStruct((D, B), jnp.float32),
        jax.ShapeDtypeStruct((1, B), jnp.float32),
        jax.ShapeDtypeStruct((1, B), jnp.float32),
    ],
)

_tc2_call = pl.pallas_call(
    _tc2_body,
    grid=(B // BB,),
    in_specs=[
        _col(SPB),
        _col(N_DENSE),
        _col(256),
        _col(D),
        _col(1),
        _col(1),
        _full((SP + N_DENSE, 256)),
        _full((256, 1)),
        _full((256, 128)),
        _full((128, 1)),
        _full((64, 128)),
        _full((64, 1)),
        _full((1, 64)),
        _full((1, 1)),
        _full((1, SP + N_DENSE)),
        _full((1, 1)),
    ],
    out_specs=_col(1),
    out_shape=jax.ShapeDtypeStruct((1, B), jnp.float32),
)


def kernel(dense_input, sparse_input, embed_tables, W_fm, b_fm,
           W1, b1, W2, b2, W3, b3, Wd, bd):
    tableT = embed_tables.transpose(0, 2, 1).reshape(SP, V)
    idxT = sparse_input.T
    wfmt = W_fm.T
    xsA = _gather(0, SPA)(idxT, tableT)
    xsB = _gather(SPA, SPB)(idxT, tableT)
    p1, s1a, qa, f1a = _tc1_call(xsA, W1, wfmt)
    outT = _tc2_call(
        xsB, dense_input.T, p1, s1a, qa, f1a,
        W1, b1.reshape(-1, 1),
        W2, b2.reshape(-1, 1),
        W3.T, b3.reshape(-1, 1),
        Wd.T, bd.reshape(1, 1),
        wfmt, b_fm.reshape(1, 1),
    )
    return outT.reshape(B)
